# baseline (speedup 1.0000x reference)
import functools

import jax
import jax.numpy as jnp
from jax import lax
from jax.experimental import pallas as pl
from jax.experimental.pallas import tpu as pltpu
from jax.experimental.pallas import tpu_sc as plsc


def _leaky(x):
    return jnp.where(x > 0, x, 0.01 * x)


def _mlp2_body(x_ref, w1_ref, b1_ref, w2_ref, b2_ref, o_ref):
    h = jnp.dot(x_ref[...], w1_ref[...], preferred_element_type=jnp.float32)
    h = _leaky(h + b1_ref[...])
    o_ref[...] = jnp.dot(h, w2_ref[...], preferred_element_type=jnp.float32) + b2_ref[...]


def _mlp2(x, W1, b1, W2, b2, bm):
    n, d_in = x.shape
    d_mid = W1.shape[1]
    d_out = W2.shape[1]
    return pl.pallas_call(
        _mlp2_body,
        grid=(n // bm,),
        in_specs=[
            pl.BlockSpec((bm, d_in), lambda i: (i, 0)),
            pl.BlockSpec((d_in, d_mid), lambda i: (0, 0)),
            pl.BlockSpec((1, d_mid), lambda i: (0, 0)),
            pl.BlockSpec((d_mid, d_out), lambda i: (0, 0)),
            pl.BlockSpec((1, d_out), lambda i: (0, 0)),
        ],
        out_specs=pl.BlockSpec((bm, d_out), lambda i: (i, 0)),
        out_shape=jax.ShapeDtypeStruct((n, d_out), jnp.float32),
    )(x, W1, b1.reshape(1, -1), W2, b2.reshape(1, -1))


def _mp_step_body(adj_ref, y_ref, yblk_ref, w1_ref, b1_ref, w2_ref, b2_ref, o_ref):
    ay = jnp.dot(adj_ref[...], y_ref[...], preferred_element_type=jnp.float32)
    h = _leaky(jnp.dot(ay, w1_ref[...], preferred_element_type=jnp.float32) + b1_ref[...])
    g = jnp.dot(h, w2_ref[...], preferred_element_type=jnp.float32) + b2_ref[...]
    o_ref[...] = g + yblk_ref[...]


def _mp_step(adj, y, W1, b1, W2, b2, bm):
    n, h = y.shape
    return pl.pallas_call(
        _mp_step_body,
        grid=(n // bm,),
        in_specs=[
            pl.BlockSpec((bm, n), lambda i: (i, 0)),
            pl.BlockSpec((n, h), lambda i: (0, 0)),
            pl.BlockSpec((bm, h), lambda i: (i, 0)),
            pl.BlockSpec((h, h), lambda i: (0, 0)),
            pl.BlockSpec((1, h), lambda i: (0, 0)),
            pl.BlockSpec((h, h), lambda i: (0, 0)),
            pl.BlockSpec((1, h), lambda i: (0, 0)),
        ],
        out_specs=pl.BlockSpec((bm, h), lambda i: (i, 0)),
        out_shape=jax.ShapeDtypeStruct((n, h), jnp.float32),
    )(adj, y, y, W1, b1.reshape(1, -1), W2, b2.reshape(1, -1))


def _post_ab_body(y_ref, wk11, bk11, wk12, bk12,
                  wk21, bk21, wk22, bk22, a_ref, b_ref):
    y = y_ref[...]
    ha = _leaky(jnp.dot(y, wk11[...], preferred_element_type=jnp.float32) + bk11[...])
    a_ref[...] = jnp.dot(ha, wk12[...], preferred_element_type=jnp.float32) + bk12[...]
    hb = _leaky(jnp.dot(y, wk21[...], preferred_element_type=jnp.float32) + bk21[...])
    b_ref[...] = jnp.dot(hb, wk22[...], preferred_element_type=jnp.float32) + bk22[...]


def _node_post_ab(y, wk11, bk11, wk12q, bk12q, wk21, bk21, wk22q, bk22q, bm):
    n, h = y.shape
    wspec = pl.BlockSpec((h, h), lambda i: (0, 0))
    bspec = pl.BlockSpec((1, h), lambda i: (0, 0))
    return pl.pallas_call(
        _post_ab_body,
        grid=(n // bm,),
        in_specs=[
            pl.BlockSpec((bm, h), lambda i: (i, 0)),
            wspec, bspec, wspec, bspec,
            wspec, bspec, wspec, bspec,
        ],
        out_specs=[
            pl.BlockSpec((bm, h), lambda i: (i, 0)),
            pl.BlockSpec((bm, h), lambda i: (i, 0)),
        ],
        out_shape=[
            jax.ShapeDtypeStruct((n, h), jnp.float32),
            jax.ShapeDtypeStruct((n, h), jnp.float32),
        ],
    )(y, wk11, bk11.reshape(1, -1), wk12q, bk12q.reshape(1, -1),
      wk21, bk21.reshape(1, -1), wk22q, bk22q.reshape(1, -1))


def _edge_body(s_ref, ef_ref, wl1, bl1, wl2, blc, wq2, bq2, o_ref):
    hl = _leaky(jnp.dot(ef_ref[...], wl1[...], preferred_element_type=jnp.float32) + bl1[...])
    c = jnp.dot(hl, wl2[...], preferred_element_type=jnp.float32) + blc[...]
    t = _leaky(s_ref[...] + c)
    o_ref[...] = jnp.dot(t, wq2[...], preferred_element_type=jnp.float32) + bq2[...]


def _edge_final(S, ef, wl1, bl1, wl2q, blc, wq2, bq2, bm, blk_off=0):
    e, h = S.shape
    e_feats = ef.shape[1]
    e_out = wq2.shape[1]
    return pl.pallas_call(
        _edge_body,
        grid=(e // bm,),
        in_specs=[
            pl.BlockSpec((bm, h), lambda i: (i, 0)),
            pl.BlockSpec((bm, e_feats), lambda i: (i + blk_off, 0)),
            pl.BlockSpec((e_feats, h), lambda i: (0, 0)),
            pl.BlockSpec((1, h), lambda i: (0, 0)),
            pl.BlockSpec((h, h), lambda i: (0, 0)),
            pl.BlockSpec((1, h), lambda i: (0, 0)),
            pl.BlockSpec((h, e_out), lambda i: (0, 0)),
            pl.BlockSpec((1, e_out), lambda i: (0, 0)),
        ],
        out_specs=pl.BlockSpec((bm, e_out), lambda i: (i, 0)),
        out_shape=jax.ShapeDtypeStruct((e, e_out), jnp.float32),
    )(S, ef, wl1, bl1.reshape(1, -1), wl2q, blc.reshape(1, -1),
      wq2, bq2.reshape(1, -1))


_NC, _NS = 2, 16
_NW = _NC * _NS


def _make_gather_combine(n, h, e, ch, eoff=0, nbuf=3):
    per_w = e // _NW
    n_chunks = per_w // ch
    assert per_w % ch == 0 and ch % 8 == 0
    mesh = plsc.VectorSubcoreMesh(core_axis_name="c", subcore_axis_name="s")

    @functools.partial(
        pl.kernel,
        mesh=mesh,
        compiler_params=pltpu.CompilerParams(use_tc_tiling_on_sc=False),
        out_type=jax.ShapeDtypeStruct((e, h), jnp.float32),
        scratch_types=[
            pltpu.VMEM((per_w,), jnp.int32),
            pltpu.VMEM((per_w,), jnp.int32),
            pltpu.VMEM((nbuf, ch, h), jnp.float32),
            pltpu.SemaphoreType.DMA((nbuf,)),
            pltpu.SemaphoreType.DMA((nbuf,)),
            pltpu.SemaphoreType.DMA((nbuf,)),
        ],
    )
    def gather_combine(a_hbm, b_hbm, e0_hbm, e1_hbm, s_hbm,
                       idx0, idx1, buf, gsem, bsem, ssem):
        wid = lax.axis_index("s") * _NC + lax.axis_index("c")
        wbase = wid * per_w
        pltpu.sync_copy(e0_hbm.at[pl.ds(eoff + wbase, per_w)], idx0)
        pltpu.sync_copy(e1_hbm.at[pl.ds(eoff + wbase, per_w)], idx1)

        def fire(c, p):
            pltpu.async_copy(a_hbm.at[idx0.at[pl.ds(c * ch, ch)]],
                             buf.at[p], gsem.at[p])

        for p in range(min(nbuf, n_chunks)):
            fire(p, p)
        for c in range(n_chunks):
            p = c % nbuf
            dst = s_hbm.at[pl.ds(wbase + c * ch, ch)]
            pltpu.make_async_copy(a_hbm.at[idx0.at[pl.ds(c * ch, ch)]],
                                  buf.at[p], gsem.at[p]).wait()
            pltpu.async_copy(b_hbm.at[idx1.at[pl.ds(c * ch, ch)]],
                             buf.at[p], bsem.at[p], add=True).wait()
            pltpu.async_copy(buf.at[p], dst, ssem.at[p])
            if c + nbuf < n_chunks:
                pltpu.make_async_copy(buf.at[p], dst, ssem.at[p]).wait()
                fire(c + nbuf, p)
        for c in range(max(0, n_chunks - nbuf), n_chunks):
            p = c % nbuf
            pltpu.make_async_copy(
                buf.at[p], s_hbm.at[pl.ds(wbase + c * ch, ch)],
                ssem.at[p]).wait()

    return gather_combine


def kernel(node_feats, adj_mat, edges, edge_feats, params):
    (wf1, bf1), (wf2, bf2) = params['f']
    (wg1, bg1), (wg2, bg2) = params['g']
    (wh1, bh1), (wh2, bh2) = params['h']
    (wk11, bk11), (wk12, bk12) = params['k1']
    (wk21, bk21), (wk22, bk22) = params['k2']
    (wl1, bl1), (wl2, bl2) = params['l']
    (wq1, bq1), (wq2, bq2) = params['q']

    n = node_feats.shape[0]
    e = edges.shape[0]
    h = wf2.shape[1]

    wk12q = wk12 @ wq1
    bk12q = bk12 @ wq1
    wk22q = wk22 @ wq1
    bk22q = bk22 @ wq1
    wl2q = wl2 @ wq1
    blc = bl2 @ wq1 + bq1

    bm_n = 1000 if n % 1000 == 0 else 8
    bm_mp = 400 if n % 400 == 0 else 8
    bm_e = 2000 if e % 2000 == 0 else 8

    y = _mlp2(node_feats, wf1, bf1, wf2, bf2, bm_n)
    for _ in range(2):
        y = _mp_step(adj_mat, y, wg1, bg1, wg2, bg2, bm_mp)
    a_tab, b_tab = _node_post_ab(
        y, wk11, bk11, wk12q, bk12q, wk21, bk21, wk22q, bk22q, bm_n)

    e0 = jnp.asarray(edges[:, 0], jnp.int32)
    e1 = jnp.asarray(edges[:, 1], jnp.int32)
    n_split = 2 if e % 2 == 0 else 1
    e_part = e // n_split
    ch = next((c for c in (400, 200, 8) if (e_part // _NW) % c == 0
               and e_part % _NW == 0), None)
    if ch is None:
        n_split, e_part = 1, e
        ch = next(c for c in (400, 200, 8) if (e // _NW) % c == 0)
    outs = []
    for si in range(n_split):
        s_i = _make_gather_combine(n, h, e_part, ch, eoff=si * e_part)(
            a_tab, b_tab, e0, e1)
        outs.append(_edge_final(s_i, edge_feats, wl1, bl1, wl2q, blc,
                                wq2, bq2, bm_e, blk_off=si * (e_part // bm_e)))

    node_outputs = _mlp2(y, wh1, bh1, wh2, bh2, bm_n)
    edge_outputs = outs[0] if n_split == 1 else jnp.concatenate(outs, axis=0)
    return (node_outputs, edge_outputs)


# speedup vs baseline: 1.0196x; 1.0196x over previous
import functools

import jax
import jax.numpy as jnp
from jax import lax
from jax.experimental import pallas as pl
from jax.experimental.pallas import tpu as pltpu
from jax.experimental.pallas import tpu_sc as plsc


def _leaky(x):
    return jnp.where(x > 0, x, 0.01 * x)


def _mlp2_body(x_ref, w1_ref, b1_ref, w2_ref, b2_ref, o_ref):
    h = jnp.dot(x_ref[...], w1_ref[...], preferred_element_type=jnp.float32)
    h = _leaky(h + b1_ref[...])
    o_ref[...] = jnp.dot(h, w2_ref[...], preferred_element_type=jnp.float32) + b2_ref[...]


def _mlp2(x, W1, b1, W2, b2, bm):
    n, d_in = x.shape
    d_mid = W1.shape[1]
    d_out = W2.shape[1]
    return pl.pallas_call(
        _mlp2_body,
        grid=(n // bm,),
        in_specs=[
            pl.BlockSpec((bm, d_in), lambda i: (i, 0)),
            pl.BlockSpec((d_in, d_mid), lambda i: (0, 0)),
            pl.BlockSpec((1, d_mid), lambda i: (0, 0)),
            pl.BlockSpec((d_mid, d_out), lambda i: (0, 0)),
            pl.BlockSpec((1, d_out), lambda i: (0, 0)),
        ],
        out_specs=pl.BlockSpec((bm, d_out), lambda i: (i, 0)),
        out_shape=jax.ShapeDtypeStruct((n, d_out), jnp.float32),
    )(x, W1, b1.reshape(1, -1), W2, b2.reshape(1, -1))


def _mp_step_body(adj_ref, y_ref, yblk_ref, w1_ref, b1_ref, w2_ref, b2_ref, o_ref):
    ay = jnp.dot(adj_ref[...], y_ref[...], preferred_element_type=jnp.float32)
    h = _leaky(jnp.dot(ay, w1_ref[...], preferred_element_type=jnp.float32) + b1_ref[...])
    g = jnp.dot(h, w2_ref[...], preferred_element_type=jnp.float32) + b2_ref[...]
    o_ref[...] = g + yblk_ref[...]


def _mp_step(adj, y, W1, b1, W2, b2, bm):
    n, h = y.shape
    return pl.pallas_call(
        _mp_step_body,
        grid=(n // bm,),
        in_specs=[
            pl.BlockSpec((bm, n), lambda i: (i, 0)),
            pl.BlockSpec((n, h), lambda i: (0, 0)),
            pl.BlockSpec((bm, h), lambda i: (i, 0)),
            pl.BlockSpec((h, h), lambda i: (0, 0)),
            pl.BlockSpec((1, h), lambda i: (0, 0)),
            pl.BlockSpec((h, h), lambda i: (0, 0)),
            pl.BlockSpec((1, h), lambda i: (0, 0)),
        ],
        out_specs=pl.BlockSpec((bm, h), lambda i: (i, 0)),
        out_shape=jax.ShapeDtypeStruct((n, h), jnp.float32),
    )(adj, y, y, W1, b1.reshape(1, -1), W2, b2.reshape(1, -1))


def _post_ab_body(y_ref, wk11, bk11, wk12, bk12,
                  wk21, bk21, wk22, bk22, a_ref, b_ref):
    y = y_ref[...]
    ha = _leaky(jnp.dot(y, wk11[...], preferred_element_type=jnp.float32) + bk11[...])
    a_ref[...] = jnp.dot(ha, wk12[...], preferred_element_type=jnp.float32) + bk12[...]
    hb = _leaky(jnp.dot(y, wk21[...], preferred_element_type=jnp.float32) + bk21[...])
    b_ref[...] = jnp.dot(hb, wk22[...], preferred_element_type=jnp.float32) + bk22[...]


def _node_post_ab(y, wk11, bk11, wk12q, bk12q, wk21, bk21, wk22q, bk22q, bm):
    n, h = y.shape
    wspec = pl.BlockSpec((h, h), lambda i: (0, 0))
    bspec = pl.BlockSpec((1, h), lambda i: (0, 0))
    return pl.pallas_call(
        _post_ab_body,
        grid=(n // bm,),
        in_specs=[
            pl.BlockSpec((bm, h), lambda i: (i, 0)),
            wspec, bspec, wspec, bspec,
            wspec, bspec, wspec, bspec,
        ],
        out_specs=[
            pl.BlockSpec((bm, h), lambda i: (i, 0)),
            pl.BlockSpec((bm, h), lambda i: (i, 0)),
        ],
        out_shape=[
            jax.ShapeDtypeStruct((n, h), jnp.float32),
            jax.ShapeDtypeStruct((n, h), jnp.float32),
        ],
    )(y, wk11, bk11.reshape(1, -1), wk12q, bk12q.reshape(1, -1),
      wk21, bk21.reshape(1, -1), wk22q, bk22q.reshape(1, -1))


def _edge_body(s_ref, ef_ref, wl1, bl1, wl2, blc, wq2, bq2, o_ref):
    hl = _leaky(jnp.dot(ef_ref[...], wl1[...], preferred_element_type=jnp.float32) + bl1[...])
    c = jnp.dot(hl, wl2[...], preferred_element_type=jnp.float32) + blc[...]
    t = _leaky(s_ref[...] + c)
    o_ref[...] = jnp.dot(t, wq2[...], preferred_element_type=jnp.float32) + bq2[...]


def _edge_final(S, ef, wl1, bl1, wl2q, blc, wq2, bq2, bm, blk_off=0):
    e, h = S.shape
    e_feats = ef.shape[1]
    e_out = wq2.shape[1]
    return pl.pallas_call(
        _edge_body,
        grid=(e // bm,),
        in_specs=[
            pl.BlockSpec((bm, h), lambda i: (i, 0)),
            pl.BlockSpec((bm, e_feats), lambda i: (i + blk_off, 0)),
            pl.BlockSpec((e_feats, h), lambda i: (0, 0)),
            pl.BlockSpec((1, h), lambda i: (0, 0)),
            pl.BlockSpec((h, h), lambda i: (0, 0)),
            pl.BlockSpec((1, h), lambda i: (0, 0)),
            pl.BlockSpec((h, e_out), lambda i: (0, 0)),
            pl.BlockSpec((1, e_out), lambda i: (0, 0)),
        ],
        out_specs=pl.BlockSpec((bm, e_out), lambda i: (i, 0)),
        out_shape=jax.ShapeDtypeStruct((e, e_out), jnp.float32),
    )(S, ef, wl1, bl1.reshape(1, -1), wl2q, blc.reshape(1, -1),
      wq2, bq2.reshape(1, -1))


_NC, _NS = 2, 16
_NW = _NC * _NS


def _make_gather_combine(n, h, e, ch, eoff=0, nbuf=3):
    per_w = e // _NW
    n_chunks = per_w // ch
    assert per_w % ch == 0 and ch % 8 == 0
    mesh = plsc.VectorSubcoreMesh(core_axis_name="c", subcore_axis_name="s")

    @functools.partial(
        pl.kernel,
        mesh=mesh,
        compiler_params=pltpu.CompilerParams(use_tc_tiling_on_sc=False),
        out_type=jax.ShapeDtypeStruct((e, h), jnp.float32),
        scratch_types=[
            pltpu.VMEM((per_w,), jnp.int32),
            pltpu.VMEM((per_w,), jnp.int32),
            pltpu.VMEM((nbuf, ch, h), jnp.float32),
            pltpu.SemaphoreType.DMA((nbuf,)),
            pltpu.SemaphoreType.DMA((nbuf,)),
            pltpu.SemaphoreType.DMA((nbuf,)),
        ],
    )
    def gather_combine(a_hbm, b_hbm, e0_hbm, e1_hbm, s_hbm,
                       idx0, idx1, buf, gsem, bsem, ssem):
        wid = lax.axis_index("s") * _NC + lax.axis_index("c")
        wbase = wid * per_w
        pltpu.sync_copy(e0_hbm.at[pl.ds(eoff + wbase, per_w)], idx0)
        pltpu.sync_copy(e1_hbm.at[pl.ds(eoff + wbase, per_w)], idx1)

        def fire(c, p):
            pltpu.async_copy(a_hbm.at[idx0.at[pl.ds(c * ch, ch)]],
                             buf.at[p], gsem.at[p])

        for p in range(min(nbuf, n_chunks)):
            fire(p, p)
        for c in range(n_chunks):
            p = c % nbuf
            dst = s_hbm.at[pl.ds(wbase + c * ch, ch)]
            pltpu.make_async_copy(a_hbm.at[idx0.at[pl.ds(c * ch, ch)]],
                                  buf.at[p], gsem.at[p]).wait()
            pltpu.async_copy(b_hbm.at[idx1.at[pl.ds(c * ch, ch)]],
                             buf.at[p], bsem.at[p], add=True).wait()
            pltpu.async_copy(buf.at[p], dst, ssem.at[p])
            if c + nbuf < n_chunks:
                pltpu.make_async_copy(buf.at[p], dst, ssem.at[p]).wait()
                fire(c + nbuf, p)
        for c in range(max(0, n_chunks - nbuf), n_chunks):
            p = c % nbuf
            pltpu.make_async_copy(
                buf.at[p], s_hbm.at[pl.ds(wbase + c * ch, ch)],
                ssem.at[p]).wait()

    return gather_combine


def kernel(node_feats, adj_mat, edges, edge_feats, params):
    (wf1, bf1), (wf2, bf2) = params['f']
    (wg1, bg1), (wg2, bg2) = params['g']
    (wh1, bh1), (wh2, bh2) = params['h']
    (wk11, bk11), (wk12, bk12) = params['k1']
    (wk21, bk21), (wk22, bk22) = params['k2']
    (wl1, bl1), (wl2, bl2) = params['l']
    (wq1, bq1), (wq2, bq2) = params['q']

    n = node_feats.shape[0]
    e = edges.shape[0]
    h = wf2.shape[1]

    wk12q = wk12 @ wq1
    bk12q = bk12 @ wq1
    wk22q = wk22 @ wq1
    bk22q = bk22 @ wq1
    wl2q = wl2 @ wq1
    blc = bl2 @ wq1 + bq1

    bm_n = 1000 if n % 1000 == 0 else 8
    bm_mp = 400 if n % 400 == 0 else 8
    bm_e = 2000 if e % 2000 == 0 else 8

    y = _mlp2(node_feats, wf1, bf1, wf2, bf2, bm_n)
    for _ in range(2):
        y = _mp_step(adj_mat, y, wg1, bg1, wg2, bg2, bm_mp)
    a_tab, b_tab = _node_post_ab(
        y, wk11, bk11, wk12q, bk12q, wk21, bk21, wk22q, bk22q, bm_n)

    e0 = jnp.asarray(edges[:, 0], jnp.int32)
    e1 = jnp.asarray(edges[:, 1], jnp.int32)
    n_split, e_part = 1, e
    ch = next(c for c in (400, 200, 8) if (e // _NW) % c == 0)
    outs = []
    for si in range(n_split):
        s_i = _make_gather_combine(n, h, e_part, ch, eoff=si * e_part)(
            a_tab, b_tab, e0, e1)
        outs.append(_edge_final(s_i, edge_feats, wl1, bl1, wl2q, blc,
                                wq2, bq2, bm_e, blk_off=si * (e_part // bm_e)))

    node_outputs = _mlp2(y, wh1, bh1, wh2, bh2, bm_n)
    edge_outputs = outs[0] if n_split == 1 else jnp.concatenate(outs, axis=0)
    return (node_outputs, edge_outputs)


# speedup vs baseline: 1.0277x; 1.0080x over previous
import functools

import jax
import jax.numpy as jnp
from jax import lax
from jax.experimental import pallas as pl
from jax.experimental.pallas import tpu as pltpu
from jax.experimental.pallas import tpu_sc as plsc


def _leaky(x):
    return jnp.where(x > 0, x, 0.01 * x)


def _mlp2_body(x_ref, w1_ref, b1_ref, w2_ref, b2_ref, o_ref):
    h = jnp.dot(x_ref[...], w1_ref[...], preferred_element_type=jnp.float32)
    h = _leaky(h + b1_ref[...])
    o_ref[...] = jnp.dot(h, w2_ref[...], preferred_element_type=jnp.float32) + b2_ref[...]


def _mlp2(x, W1, b1, W2, b2, bm):
    n, d_in = x.shape
    d_mid = W1.shape[1]
    d_out = W2.shape[1]
    return pl.pallas_call(
        _mlp2_body,
        grid=(n // bm,),
        in_specs=[
            pl.BlockSpec((bm, d_in), lambda i: (i, 0)),
            pl.BlockSpec((d_in, d_mid), lambda i: (0, 0)),
            pl.BlockSpec((1, d_mid), lambda i: (0, 0)),
            pl.BlockSpec((d_mid, d_out), lambda i: (0, 0)),
            pl.BlockSpec((1, d_out), lambda i: (0, 0)),
        ],
        out_specs=pl.BlockSpec((bm, d_out), lambda i: (i, 0)),
        out_shape=jax.ShapeDtypeStruct((n, d_out), jnp.float32),
    )(x, W1, b1.reshape(1, -1), W2, b2.reshape(1, -1))


def _mp_step_body(adj1_ref, adj2_ref, y_ref, yblk_ref, w1_ref, b1_ref,
                  w2_ref, b2_ref, o_ref):
    y = y_ref[...]
    ay1 = jnp.dot(adj1_ref[...], y, preferred_element_type=jnp.float32)
    ay2 = jnp.dot(adj2_ref[...], y, preferred_element_type=jnp.float32)
    ay = jnp.concatenate([ay1, ay2], axis=0)
    h = _leaky(jnp.dot(ay, w1_ref[...], preferred_element_type=jnp.float32) + b1_ref[...])
    g = jnp.dot(h, w2_ref[...], preferred_element_type=jnp.float32) + b2_ref[...]
    o_ref[...] = g + yblk_ref[...]


def _mp_step(adj, y, W1, b1, W2, b2, bm):
    n, h = y.shape
    hb = bm // 2
    return pl.pallas_call(
        _mp_step_body,
        grid=(n // bm,),
        in_specs=[
            pl.BlockSpec((hb, n), lambda i: (2 * i, 0)),
            pl.BlockSpec((hb, n), lambda i: (2 * i + 1, 0)),
            pl.BlockSpec((n, h), lambda i: (0, 0)),
            pl.BlockSpec((bm, h), lambda i: (i, 0)),
            pl.BlockSpec((h, h), lambda i: (0, 0)),
            pl.BlockSpec((1, h), lambda i: (0, 0)),
            pl.BlockSpec((h, h), lambda i: (0, 0)),
            pl.BlockSpec((1, h), lambda i: (0, 0)),
        ],
        out_specs=pl.BlockSpec((bm, h), lambda i: (i, 0)),
        out_shape=jax.ShapeDtypeStruct((n, h), jnp.float32),
    )(adj, adj, y, y, W1, b1.reshape(1, -1), W2, b2.reshape(1, -1))


def _post_ab_body(y_ref, wk11, bk11, wk12, bk12,
                  wk21, bk21, wk22, bk22, a_ref, b_ref):
    y = y_ref[...]
    ha = _leaky(jnp.dot(y, wk11[...], preferred_element_type=jnp.float32) + bk11[...])
    a_ref[...] = jnp.dot(ha, wk12[...], preferred_element_type=jnp.float32) + bk12[...]
    hb = _leaky(jnp.dot(y, wk21[...], preferred_element_type=jnp.float32) + bk21[...])
    b_ref[...] = jnp.dot(hb, wk22[...], preferred_element_type=jnp.float32) + bk22[...]


def _node_post_ab(y, wk11, bk11, wk12q, bk12q, wk21, bk21, wk22q, bk22q, bm):
    n, h = y.shape
    wspec = pl.BlockSpec((h, h), lambda i: (0, 0))
    bspec = pl.BlockSpec((1, h), lambda i: (0, 0))
    return pl.pallas_call(
        _post_ab_body,
        grid=(n // bm,),
        in_specs=[
            pl.BlockSpec((bm, h), lambda i: (i, 0)),
            wspec, bspec, wspec, bspec,
            wspec, bspec, wspec, bspec,
        ],
        out_specs=[
            pl.BlockSpec((bm, h), lambda i: (i, 0)),
            pl.BlockSpec((bm, h), lambda i: (i, 0)),
        ],
        out_shape=[
            jax.ShapeDtypeStruct((n, h), jnp.float32),
            jax.ShapeDtypeStruct((n, h), jnp.float32),
        ],
    )(y, wk11, bk11.reshape(1, -1), wk12q, bk12q.reshape(1, -1),
      wk21, bk21.reshape(1, -1), wk22q, bk22q.reshape(1, -1))


def _edge_body(s_ref, ef_ref, wl1, bl1, wl2, blc, wq2, bq2, o_ref):
    hl = _leaky(jnp.dot(ef_ref[...], wl1[...], preferred_element_type=jnp.float32) + bl1[...])
    c = jnp.dot(hl, wl2[...], preferred_element_type=jnp.float32) + blc[...]
    t = _leaky(s_ref[...] + c)
    o_ref[...] = jnp.dot(t, wq2[...], preferred_element_type=jnp.float32) + bq2[...]


def _edge_final(S, ef, wl1, bl1, wl2q, blc, wq2, bq2, bm, blk_off=0):
    e, h = S.shape
    e_feats = ef.shape[1]
    e_out = wq2.shape[1]
    return pl.pallas_call(
        _edge_body,
        grid=(e // bm,),
        in_specs=[
            pl.BlockSpec((bm, h), lambda i: (i, 0)),
            pl.BlockSpec((bm, e_feats), lambda i: (i + blk_off, 0)),
            pl.BlockSpec((e_feats, h), lambda i: (0, 0)),
            pl.BlockSpec((1, h), lambda i: (0, 0)),
            pl.BlockSpec((h, h), lambda i: (0, 0)),
            pl.BlockSpec((1, h), lambda i: (0, 0)),
            pl.BlockSpec((h, e_out), lambda i: (0, 0)),
            pl.BlockSpec((1, e_out), lambda i: (0, 0)),
        ],
        out_specs=pl.BlockSpec((bm, e_out), lambda i: (i, 0)),
        out_shape=jax.ShapeDtypeStruct((e, e_out), jnp.float32),
    )(S, ef, wl1, bl1.reshape(1, -1), wl2q, blc.reshape(1, -1),
      wq2, bq2.reshape(1, -1))


_NC, _NS = 2, 16
_NW = _NC * _NS


def _make_gather_combine(n, h, e, ch, eoff=0, nbuf=3):
    per_w = e // _NW
    n_chunks = per_w // ch
    assert per_w % ch == 0 and ch % 8 == 0
    mesh = plsc.VectorSubcoreMesh(core_axis_name="c", subcore_axis_name="s")

    @functools.partial(
        pl.kernel,
        mesh=mesh,
        compiler_params=pltpu.CompilerParams(use_tc_tiling_on_sc=False),
        out_type=jax.ShapeDtypeStruct((e, h), jnp.float32),
        scratch_types=[
            pltpu.VMEM((per_w,), jnp.int32),
            pltpu.VMEM((per_w,), jnp.int32),
            pltpu.VMEM((nbuf, ch, h), jnp.float32),
            pltpu.SemaphoreType.DMA((nbuf,)),
            pltpu.SemaphoreType.DMA((nbuf,)),
            pltpu.SemaphoreType.DMA((nbuf,)),
        ],
    )
    def gather_combine(a_hbm, b_hbm, e0_hbm, e1_hbm, s_hbm,
                       idx0, idx1, buf, gsem, bsem, ssem):
        wid = lax.axis_index("s") * _NC + lax.axis_index("c")
        wbase = wid * per_w
        pltpu.sync_copy(e0_hbm.at[pl.ds(eoff + wbase, per_w)], idx0)
        pltpu.sync_copy(e1_hbm.at[pl.ds(eoff + wbase, per_w)], idx1)

        def fire(c, p):
            pltpu.async_copy(a_hbm.at[idx0.at[pl.ds(c * ch, ch)]],
                             buf.at[p], gsem.at[p])

        for p in range(min(nbuf, n_chunks)):
            fire(p, p)
        for c in range(n_chunks):
            p = c % nbuf
            dst = s_hbm.at[pl.ds(wbase + c * ch, ch)]
            pltpu.make_async_copy(a_hbm.at[idx0.at[pl.ds(c * ch, ch)]],
                                  buf.at[p], gsem.at[p]).wait()
            pltpu.async_copy(b_hbm.at[idx1.at[pl.ds(c * ch, ch)]],
                             buf.at[p], bsem.at[p], add=True).wait()
            pltpu.async_copy(buf.at[p], dst, ssem.at[p])
            if c + nbuf < n_chunks:
                pltpu.make_async_copy(buf.at[p], dst, ssem.at[p]).wait()
                fire(c + nbuf, p)
        for c in range(max(0, n_chunks - nbuf), n_chunks):
            p = c % nbuf
            pltpu.make_async_copy(
                buf.at[p], s_hbm.at[pl.ds(wbase + c * ch, ch)],
                ssem.at[p]).wait()

    return gather_combine


def kernel(node_feats, adj_mat, edges, edge_feats, params):
    (wf1, bf1), (wf2, bf2) = params['f']
    (wg1, bg1), (wg2, bg2) = params['g']
    (wh1, bh1), (wh2, bh2) = params['h']
    (wk11, bk11), (wk12, bk12) = params['k1']
    (wk21, bk21), (wk22, bk22) = params['k2']
    (wl1, bl1), (wl2, bl2) = params['l']
    (wq1, bq1), (wq2, bq2) = params['q']

    n = node_feats.shape[0]
    e = edges.shape[0]
    h = wf2.shape[1]

    wk12q = wk12 @ wq1
    bk12q = bk12 @ wq1
    wk22q = wk22 @ wq1
    bk22q = bk22 @ wq1
    wl2q = wl2 @ wq1
    blc = bl2 @ wq1 + bq1

    bm_n = 1000 if n % 1000 == 0 else 8
    bm_mp = 400 if n % 400 == 0 else 8
    bm_e = 2000 if e % 2000 == 0 else 8

    y = _mlp2(node_feats, wf1, bf1, wf2, bf2, bm_n)
    for _ in range(2):
        y = _mp_step(adj_mat, y, wg1, bg1, wg2, bg2, bm_mp)
    a_tab, b_tab = _node_post_ab(
        y, wk11, bk11, wk12q, bk12q, wk21, bk21, wk22q, bk22q, bm_n)

    e0 = jnp.asarray(edges[:, 0], jnp.int32)
    e1 = jnp.asarray(edges[:, 1], jnp.int32)
    n_split, e_part = 1, e
    ch = next(c for c in (400, 200, 8) if (e // _NW) % c == 0)
    outs = []
    for si in range(n_split):
        s_i = _make_gather_combine(n, h, e_part, ch, eoff=si * e_part)(
            a_tab, b_tab, e0, e1)
        outs.append(_edge_final(s_i, edge_feats, wl1, bl1, wl2q, blc,
                                wq2, bq2, bm_e, blk_off=si * (e_part // bm_e)))

    node_outputs = _mlp2(y, wh1, bh1, wh2, bh2, bm_n)
    edge_outputs = outs[0] if n_split == 1 else jnp.concatenate(outs, axis=0)
    return (node_outputs, edge_outputs)


# speedup vs baseline: 1.0284x; 1.0006x over previous
import functools

import jax
import jax.numpy as jnp
from jax import lax
from jax.experimental import pallas as pl
from jax.experimental.pallas import tpu as pltpu
from jax.experimental.pallas import tpu_sc as plsc


def _leaky(x):
    return jnp.where(x > 0, x, 0.01 * x)


def _mlp2_body(x_ref, w1_ref, b1_ref, w2_ref, b2_ref, o_ref):
    h = jnp.dot(x_ref[...], w1_ref[...], preferred_element_type=jnp.float32)
    h = _leaky(h + b1_ref[...])
    o_ref[...] = jnp.dot(h, w2_ref[...], preferred_element_type=jnp.float32) + b2_ref[...]


def _mlp2(x, W1, b1, W2, b2, bm):
    n, d_in = x.shape
    d_mid = W1.shape[1]
    d_out = W2.shape[1]
    return pl.pallas_call(
        _mlp2_body,
        grid=(n // bm,),
        in_specs=[
            pl.BlockSpec((bm, d_in), lambda i: (i, 0)),
            pl.BlockSpec((d_in, d_mid), lambda i: (0, 0)),
            pl.BlockSpec((1, d_mid), lambda i: (0, 0)),
            pl.BlockSpec((d_mid, d_out), lambda i: (0, 0)),
            pl.BlockSpec((1, d_out), lambda i: (0, 0)),
        ],
        out_specs=pl.BlockSpec((bm, d_out), lambda i: (i, 0)),
        out_shape=jax.ShapeDtypeStruct((n, d_out), jnp.float32),
    )(x, W1, b1.reshape(1, -1), W2, b2.reshape(1, -1))


_MP_STREAMS = 2


def _mp_step_body(*refs):
    adj_refs = refs[:_MP_STREAMS]
    y_ref, yblk_ref, w1_ref, b1_ref, w2_ref, b2_ref, o_ref = refs[_MP_STREAMS:]
    y = y_ref[...]
    ay = jnp.concatenate(
        [jnp.dot(a[...], y, preferred_element_type=jnp.float32)
         for a in adj_refs], axis=0)
    h = _leaky(jnp.dot(ay, w1_ref[...], preferred_element_type=jnp.float32) + b1_ref[...])
    g = jnp.dot(h, w2_ref[...], preferred_element_type=jnp.float32) + b2_ref[...]
    o_ref[...] = g + yblk_ref[...]


def _mp_step(adj, y, W1, b1, W2, b2, bm):
    n, h = y.shape
    ns = _MP_STREAMS
    hb = bm // ns

    def _adj_spec(j):
        return pl.BlockSpec((hb, n), lambda i: (ns * i + j, 0))

    return pl.pallas_call(
        _mp_step_body,
        grid=(n // bm,),
        in_specs=[_adj_spec(j) for j in range(ns)] + [
            pl.BlockSpec((n, h), lambda i: (0, 0)),
            pl.BlockSpec((bm, h), lambda i: (i, 0)),
            pl.BlockSpec((h, h), lambda i: (0, 0)),
            pl.BlockSpec((1, h), lambda i: (0, 0)),
            pl.BlockSpec((h, h), lambda i: (0, 0)),
            pl.BlockSpec((1, h), lambda i: (0, 0)),
        ],
        out_specs=pl.BlockSpec((bm, h), lambda i: (i, 0)),
        out_shape=jax.ShapeDtypeStruct((n, h), jnp.float32),
    )(*([adj] * ns), y, y, W1, b1.reshape(1, -1), W2, b2.reshape(1, -1))


def _post_ab_body(y_ref, wk11, bk11, wk12, bk12,
                  wk21, bk21, wk22, bk22, a_ref, b_ref):
    y = y_ref[...]
    ha = _leaky(jnp.dot(y, wk11[...], preferred_element_type=jnp.float32) + bk11[...])
    a_ref[...] = jnp.dot(ha, wk12[...], preferred_element_type=jnp.float32) + bk12[...]
    hb = _leaky(jnp.dot(y, wk21[...], preferred_element_type=jnp.float32) + bk21[...])
    b_ref[...] = jnp.dot(hb, wk22[...], preferred_element_type=jnp.float32) + bk22[...]


def _node_post_ab(y, wk11, bk11, wk12q, bk12q, wk21, bk21, wk22q, bk22q, bm):
    n, h = y.shape
    wspec = pl.BlockSpec((h, h), lambda i: (0, 0))
    bspec = pl.BlockSpec((1, h), lambda i: (0, 0))
    return pl.pallas_call(
        _post_ab_body,
        grid=(n // bm,),
        in_specs=[
            pl.BlockSpec((bm, h), lambda i: (i, 0)),
            wspec, bspec, wspec, bspec,
            wspec, bspec, wspec, bspec,
        ],
        out_specs=[
            pl.BlockSpec((bm, h), lambda i: (i, 0)),
            pl.BlockSpec((bm, h), lambda i: (i, 0)),
        ],
        out_shape=[
            jax.ShapeDtypeStruct((n, h), jnp.float32),
            jax.ShapeDtypeStruct((n, h), jnp.float32),
        ],
    )(y, wk11, bk11.reshape(1, -1), wk12q, bk12q.reshape(1, -1),
      wk21, bk21.reshape(1, -1), wk22q, bk22q.reshape(1, -1))


def _edge_body(s_ref, ef_ref, wl1, bl1, wl2, blc, wq2, bq2, o_ref):
    hl = _leaky(jnp.dot(ef_ref[...], wl1[...], preferred_element_type=jnp.float32) + bl1[...])
    c = jnp.dot(hl, wl2[...], preferred_element_type=jnp.float32) + blc[...]
    t = _leaky(s_ref[...] + c)
    o_ref[...] = jnp.dot(t, wq2[...], preferred_element_type=jnp.float32) + bq2[...]


def _edge_final(S, ef, wl1, bl1, wl2q, blc, wq2, bq2, bm, blk_off=0):
    e, h = S.shape
    e_feats = ef.shape[1]
    e_out = wq2.shape[1]
    return pl.pallas_call(
        _edge_body,
        grid=(e // bm,),
        in_specs=[
            pl.BlockSpec((bm, h), lambda i: (i, 0)),
            pl.BlockSpec((bm, e_feats), lambda i: (i + blk_off, 0)),
            pl.BlockSpec((e_feats, h), lambda i: (0, 0)),
            pl.BlockSpec((1, h), lambda i: (0, 0)),
            pl.BlockSpec((h, h), lambda i: (0, 0)),
            pl.BlockSpec((1, h), lambda i: (0, 0)),
            pl.BlockSpec((h, e_out), lambda i: (0, 0)),
            pl.BlockSpec((1, e_out), lambda i: (0, 0)),
        ],
        out_specs=pl.BlockSpec((bm, e_out), lambda i: (i, 0)),
        out_shape=jax.ShapeDtypeStruct((e, e_out), jnp.float32),
    )(S, ef, wl1, bl1.reshape(1, -1), wl2q, blc.reshape(1, -1),
      wq2, bq2.reshape(1, -1))


_NC, _NS = 2, 16
_NW = _NC * _NS


def _make_gather_combine(n, h, e, ch, eoff=0, nbuf=3):
    per_w = e // _NW
    n_chunks = per_w // ch
    assert per_w % ch == 0 and ch % 8 == 0
    mesh = plsc.VectorSubcoreMesh(core_axis_name="c", subcore_axis_name="s")

    @functools.partial(
        pl.kernel,
        mesh=mesh,
        compiler_params=pltpu.CompilerParams(use_tc_tiling_on_sc=False),
        out_type=jax.ShapeDtypeStruct((e, h), jnp.float32),
        scratch_types=[
            pltpu.VMEM((per_w,), jnp.int32),
            pltpu.VMEM((per_w,), jnp.int32),
            pltpu.VMEM((nbuf, ch, h), jnp.float32),
            pltpu.SemaphoreType.DMA((nbuf,)),
            pltpu.SemaphoreType.DMA((nbuf,)),
            pltpu.SemaphoreType.DMA((nbuf,)),
        ],
    )
    def gather_combine(a_hbm, b_hbm, e0_hbm, e1_hbm, s_hbm,
                       idx0, idx1, buf, gsem, bsem, ssem):
        wid = lax.axis_index("s") * _NC + lax.axis_index("c")
        wbase = wid * per_w
        pltpu.sync_copy(e0_hbm.at[pl.ds(eoff + wbase, per_w)], idx0)
        pltpu.sync_copy(e1_hbm.at[pl.ds(eoff + wbase, per_w)], idx1)

        def fire(c, p):
            pltpu.async_copy(a_hbm.at[idx0.at[pl.ds(c * ch, ch)]],
                             buf.at[p], gsem.at[p])

        for p in range(min(nbuf, n_chunks)):
            fire(p, p)
        for c in range(n_chunks):
            p = c % nbuf
            dst = s_hbm.at[pl.ds(wbase + c * ch, ch)]
            pltpu.make_async_copy(a_hbm.at[idx0.at[pl.ds(c * ch, ch)]],
                                  buf.at[p], gsem.at[p]).wait()
            pltpu.async_copy(b_hbm.at[idx1.at[pl.ds(c * ch, ch)]],
                             buf.at[p], bsem.at[p], add=True).wait()
            pltpu.async_copy(buf.at[p], dst, ssem.at[p])
            if c + nbuf < n_chunks:
                pltpu.make_async_copy(buf.at[p], dst, ssem.at[p]).wait()
                fire(c + nbuf, p)
        for c in range(max(0, n_chunks - nbuf), n_chunks):
            p = c % nbuf
            pltpu.make_async_copy(
                buf.at[p], s_hbm.at[pl.ds(wbase + c * ch, ch)],
                ssem.at[p]).wait()

    return gather_combine


def kernel(node_feats, adj_mat, edges, edge_feats, params):
    (wf1, bf1), (wf2, bf2) = params['f']
    (wg1, bg1), (wg2, bg2) = params['g']
    (wh1, bh1), (wh2, bh2) = params['h']
    (wk11, bk11), (wk12, bk12) = params['k1']
    (wk21, bk21), (wk22, bk22) = params['k2']
    (wl1, bl1), (wl2, bl2) = params['l']
    (wq1, bq1), (wq2, bq2) = params['q']

    n = node_feats.shape[0]
    e = edges.shape[0]
    h = wf2.shape[1]

    wk12q = wk12 @ wq1
    bk12q = bk12 @ wq1
    wk22q = wk22 @ wq1
    bk22q = bk22 @ wq1
    wl2q = wl2 @ wq1
    blc = bl2 @ wq1 + bq1

    bm_n = 1000 if n % 1000 == 0 else 8
    bm_mp = 400 if n % 400 == 0 else 8 * _MP_STREAMS
    bm_e = 2000 if e % 2000 == 0 else 8

    y = _mlp2(node_feats, wf1, bf1, wf2, bf2, bm_n)
    for _ in range(2):
        y = _mp_step(adj_mat, y, wg1, bg1, wg2, bg2, bm_mp)
    a_tab, b_tab = _node_post_ab(
        y, wk11, bk11, wk12q, bk12q, wk21, bk21, wk22q, bk22q, bm_n)

    e0 = jnp.asarray(edges[:, 0], jnp.int32)
    e1 = jnp.asarray(edges[:, 1], jnp.int32)
    n_split, e_part = 1, e
    ch = next(c for c in (400, 200, 8) if (e // _NW) % c == 0)
    outs = []
    for si in range(n_split):
        s_i = _make_gather_combine(n, h, e_part, ch, eoff=si * e_part)(
            a_tab, b_tab, e0, e1)
        outs.append(_edge_final(s_i, edge_feats, wl1, bl1, wl2q, blc,
                                wq2, bq2, bm_e, blk_off=si * (e_part // bm_e)))

    node_outputs = _mlp2(y, wh1, bh1, wh2, bh2, bm_n)
    edge_outputs = outs[0] if n_split == 1 else jnp.concatenate(outs, axis=0)
    return (node_outputs, edge_outputs)


# speedup vs baseline: 1.0923x; 1.0622x over previous
import functools

import jax
import jax.numpy as jnp
from jax import lax
from jax.experimental import pallas as pl
from jax.experimental.pallas import tpu as pltpu
from jax.experimental.pallas import tpu_sc as plsc


def _leaky(x):
    return jnp.where(x > 0, x, 0.01 * x)


def _mlp2_body(x_ref, w1_ref, b1_ref, w2_ref, b2_ref, o_ref):
    h = jnp.dot(x_ref[...], w1_ref[...], preferred_element_type=jnp.float32)
    h = _leaky(h + b1_ref[...])
    o_ref[...] = jnp.dot(h, w2_ref[...], preferred_element_type=jnp.float32) + b2_ref[...]


def _mlp2(x, W1, b1, W2, b2, bm):
    n, d_in = x.shape
    d_mid = W1.shape[1]
    d_out = W2.shape[1]
    return pl.pallas_call(
        _mlp2_body,
        grid=(n // bm,),
        in_specs=[
            pl.BlockSpec((bm, d_in), lambda i: (i, 0)),
            pl.BlockSpec((d_in, d_mid), lambda i: (0, 0)),
            pl.BlockSpec((1, d_mid), lambda i: (0, 0)),
            pl.BlockSpec((d_mid, d_out), lambda i: (0, 0)),
            pl.BlockSpec((1, d_out), lambda i: (0, 0)),
        ],
        out_specs=pl.BlockSpec((bm, d_out), lambda i: (i, 0)),
        out_shape=jax.ShapeDtypeStruct((n, d_out), jnp.float32),
    )(x, W1, b1.reshape(1, -1), W2, b2.reshape(1, -1))


_MP_STREAMS = 2


def _mp_step_body(*refs):
    adj_refs = refs[:_MP_STREAMS]
    y_ref, yblk_ref, w1_ref, b1_ref, w2_ref, b2_ref, o_ref = refs[_MP_STREAMS:]
    y = y_ref[...]
    ay = jnp.concatenate(
        [jnp.dot(a[...], y, preferred_element_type=jnp.float32)
         for a in adj_refs], axis=0)
    h = _leaky(jnp.dot(ay, w1_ref[...], preferred_element_type=jnp.float32) + b1_ref[...])
    g = jnp.dot(h, w2_ref[...], preferred_element_type=jnp.float32) + b2_ref[...]
    o_ref[...] = g + yblk_ref[...]


def _mp_step(adj, y, W1, b1, W2, b2, bm):
    n, h = y.shape
    ns = _MP_STREAMS
    hb = bm // ns

    def _adj_spec(j):
        return pl.BlockSpec((hb, n), lambda i: (ns * i + j, 0))

    return pl.pallas_call(
        _mp_step_body,
        grid=(n // bm,),
        in_specs=[_adj_spec(j) for j in range(ns)] + [
            pl.BlockSpec((n, h), lambda i: (0, 0)),
            pl.BlockSpec((bm, h), lambda i: (i, 0)),
            pl.BlockSpec((h, h), lambda i: (0, 0)),
            pl.BlockSpec((1, h), lambda i: (0, 0)),
            pl.BlockSpec((h, h), lambda i: (0, 0)),
            pl.BlockSpec((1, h), lambda i: (0, 0)),
        ],
        out_specs=pl.BlockSpec((bm, h), lambda i: (i, 0)),
        out_shape=jax.ShapeDtypeStruct((n, h), jnp.float32),
    )(*([adj] * ns), y, y, W1, b1.reshape(1, -1), W2, b2.reshape(1, -1))


def _post_ab_body(y_ref, wk11, bk11, wk12, bk12,
                  wk21, bk21, wk22, bk22, a_ref, b_ref):
    y = y_ref[...]
    ha = _leaky(jnp.dot(y, wk11[...], preferred_element_type=jnp.float32) + bk11[...])
    a_ref[...] = jnp.dot(ha, wk12[...], preferred_element_type=jnp.float32) + bk12[...]
    hb = _leaky(jnp.dot(y, wk21[...], preferred_element_type=jnp.float32) + bk21[...])
    b_ref[...] = jnp.dot(hb, wk22[...], preferred_element_type=jnp.float32) + bk22[...]


def _node_post_ab(y, wk11, bk11, wk12q, bk12q, wk21, bk21, wk22q, bk22q, bm):
    n, h = y.shape
    wspec = pl.BlockSpec((h, h), lambda i: (0, 0))
    bspec = pl.BlockSpec((1, h), lambda i: (0, 0))
    return pl.pallas_call(
        _post_ab_body,
        grid=(n // bm,),
        in_specs=[
            pl.BlockSpec((bm, h), lambda i: (i, 0)),
            wspec, bspec, wspec, bspec,
            wspec, bspec, wspec, bspec,
        ],
        out_specs=[
            pl.BlockSpec((bm, h), lambda i: (i, 0)),
            pl.BlockSpec((bm, h), lambda i: (i, 0)),
        ],
        out_shape=[
            jax.ShapeDtypeStruct((n, h), jnp.float32),
            jax.ShapeDtypeStruct((n, h), jnp.float32),
        ],
    )(y, wk11, bk11.reshape(1, -1), wk12q, bk12q.reshape(1, -1),
      wk21, bk21.reshape(1, -1), wk22q, bk22q.reshape(1, -1))


def _edge_body(s_ref, ef_ref, wl1, bl1, wl2, blc, wq2, bq2, o_ref):
    hl = _leaky(jnp.dot(ef_ref[...], wl1[...], preferred_element_type=jnp.float32) + bl1[...])
    c = jnp.dot(hl, wl2[...], preferred_element_type=jnp.float32) + blc[...]
    t = _leaky(s_ref[...] + c)
    o_ref[...] = jnp.dot(t, wq2[...], preferred_element_type=jnp.float32) + bq2[...]


def _edge_final(S, ef, wl1, bl1, wl2q, blc, wq2, bq2, bm, blk_off=0):
    e, h = S.shape
    e_feats = ef.shape[1]
    e_out = wq2.shape[1]
    return pl.pallas_call(
        _edge_body,
        grid=(e // bm,),
        in_specs=[
            pl.BlockSpec((bm, h), lambda i: (i, 0)),
            pl.BlockSpec((bm, e_feats), lambda i: (i + blk_off, 0)),
            pl.BlockSpec((e_feats, h), lambda i: (0, 0)),
            pl.BlockSpec((1, h), lambda i: (0, 0)),
            pl.BlockSpec((h, h), lambda i: (0, 0)),
            pl.BlockSpec((1, h), lambda i: (0, 0)),
            pl.BlockSpec((h, e_out), lambda i: (0, 0)),
            pl.BlockSpec((1, e_out), lambda i: (0, 0)),
        ],
        out_specs=pl.BlockSpec((bm, e_out), lambda i: (i, 0)),
        out_shape=jax.ShapeDtypeStruct((e, e_out), jnp.float32),
    )(S, ef, wl1, bl1.reshape(1, -1), wl2q, blc.reshape(1, -1),
      wq2, bq2.reshape(1, -1))


_NC, _NS = 2, 16
_NW = _NC * _NS


def _make_gather_combine(n, h, e, ch, eoff=0, nbuf=4):
    per_w = e // _NW
    n_chunks = per_w // ch
    assert per_w % ch == 0 and ch % 8 == 0
    mesh = plsc.VectorSubcoreMesh(core_axis_name="c", subcore_axis_name="s")

    @functools.partial(
        pl.kernel,
        mesh=mesh,
        compiler_params=pltpu.CompilerParams(use_tc_tiling_on_sc=False),
        out_type=jax.ShapeDtypeStruct((e, h), jnp.float32),
        scratch_types=[
            pltpu.VMEM((per_w,), jnp.int32),
            pltpu.VMEM((per_w,), jnp.int32),
            pltpu.VMEM((nbuf, ch, h), jnp.float32),
            pltpu.SemaphoreType.DMA((nbuf,)),
            pltpu.SemaphoreType.DMA((nbuf,)),
            pltpu.SemaphoreType.DMA((nbuf,)),
        ],
    )
    def gather_combine(a_hbm, b_hbm, e0_hbm, e1_hbm, s_hbm,
                       idx0, idx1, buf, gsem, bsem, ssem):
        wid = lax.axis_index("s") * _NC + lax.axis_index("c")
        wbase = wid * per_w
        pltpu.sync_copy(e0_hbm.at[pl.ds(eoff + wbase, per_w)], idx0)
        pltpu.sync_copy(e1_hbm.at[pl.ds(eoff + wbase, per_w)], idx1)

        def fire(c, p):
            pltpu.async_copy(a_hbm.at[idx0.at[pl.ds(c * ch, ch)]],
                             buf.at[p], gsem.at[p])

        for p in range(min(nbuf, n_chunks)):
            fire(p, p)
        for c in range(n_chunks):
            p = c % nbuf
            dst = s_hbm.at[pl.ds(wbase + c * ch, ch)]
            pltpu.make_async_copy(a_hbm.at[idx0.at[pl.ds(c * ch, ch)]],
                                  buf.at[p], gsem.at[p]).wait()
            pltpu.async_copy(b_hbm.at[idx1.at[pl.ds(c * ch, ch)]],
                             buf.at[p], bsem.at[p], add=True).wait()
            pltpu.async_copy(buf.at[p], dst, ssem.at[p])
            if c + nbuf < n_chunks:
                pltpu.make_async_copy(buf.at[p], dst, ssem.at[p]).wait()
                fire(c + nbuf, p)
        for c in range(max(0, n_chunks - nbuf), n_chunks):
            p = c % nbuf
            pltpu.make_async_copy(
                buf.at[p], s_hbm.at[pl.ds(wbase + c * ch, ch)],
                ssem.at[p]).wait()

    return gather_combine


def kernel(node_feats, adj_mat, edges, edge_feats, params):
    (wf1, bf1), (wf2, bf2) = params['f']
    (wg1, bg1), (wg2, bg2) = params['g']
    (wh1, bh1), (wh2, bh2) = params['h']
    (wk11, bk11), (wk12, bk12) = params['k1']
    (wk21, bk21), (wk22, bk22) = params['k2']
    (wl1, bl1), (wl2, bl2) = params['l']
    (wq1, bq1), (wq2, bq2) = params['q']

    n = node_feats.shape[0]
    e = edges.shape[0]
    h = wf2.shape[1]

    wk12q = wk12 @ wq1
    bk12q = bk12 @ wq1
    wk22q = wk22 @ wq1
    bk22q = bk22 @ wq1
    wl2q = wl2 @ wq1
    blc = bl2 @ wq1 + bq1

    bm_n = 1000 if n % 1000 == 0 else 8
    bm_mp = 400 if n % 400 == 0 else 8 * _MP_STREAMS
    bm_e = 4000 if e % 4000 == 0 else 8

    y = _mlp2(node_feats, wf1, bf1, wf2, bf2, bm_n)
    for _ in range(2):
        y = _mp_step(adj_mat, y, wg1, bg1, wg2, bg2, bm_mp)
    a_tab, b_tab = _node_post_ab(
        y, wk11, bk11, wk12q, bk12q, wk21, bk21, wk22q, bk22q, bm_n)

    e0 = jnp.asarray(edges[:, 0], jnp.int32)
    e1 = jnp.asarray(edges[:, 1], jnp.int32)
    n_split, e_part = 1, e
    ch = next(c for c in (400, 200, 8) if (e // _NW) % c == 0)
    outs = []
    for si in range(n_split):
        s_i = _make_gather_combine(n, h, e_part, ch, eoff=si * e_part)(
            a_tab, b_tab, e0, e1)
        outs.append(_edge_final(s_i, edge_feats, wl1, bl1, wl2q, blc,
                                wq2, bq2, bm_e, blk_off=si * (e_part // bm_e)))

    node_outputs = _mlp2(y, wh1, bh1, wh2, bh2, bm_n)
    edge_outputs = outs[0] if n_split == 1 else jnp.concatenate(outs, axis=0)
    return (node_outputs, edge_outputs)


# speedup vs baseline: 1.1100x; 1.0162x over previous
import functools

import jax
import jax.numpy as jnp
from jax import lax
from jax.experimental import pallas as pl
from jax.experimental.pallas import tpu as pltpu
from jax.experimental.pallas import tpu_sc as plsc


def _leaky(x):
    return jnp.where(x > 0, x, 0.01 * x)


def _mlp2_body(x_ref, w1_ref, b1_ref, w2_ref, b2_ref, o_ref):
    h = jnp.dot(x_ref[...], w1_ref[...], preferred_element_type=jnp.float32)
    h = _leaky(h + b1_ref[...])
    o_ref[...] = jnp.dot(h, w2_ref[...], preferred_element_type=jnp.float32) + b2_ref[...]


def _mlp2(x, W1, b1, W2, b2, bm):
    n, d_in = x.shape
    d_mid = W1.shape[1]
    d_out = W2.shape[1]
    return pl.pallas_call(
        _mlp2_body,
        grid=(n // bm,),
        in_specs=[
            pl.BlockSpec((bm, d_in), lambda i: (i, 0)),
            pl.BlockSpec((d_in, d_mid), lambda i: (0, 0)),
            pl.BlockSpec((1, d_mid), lambda i: (0, 0)),
            pl.BlockSpec((d_mid, d_out), lambda i: (0, 0)),
            pl.BlockSpec((1, d_out), lambda i: (0, 0)),
        ],
        out_specs=pl.BlockSpec((bm, d_out), lambda i: (i, 0)),
        out_shape=jax.ShapeDtypeStruct((n, d_out), jnp.float32),
    )(x, W1, b1.reshape(1, -1), W2, b2.reshape(1, -1))


_MP_STREAMS = 2


def _mp_step_body(*refs):
    adj_refs = refs[:_MP_STREAMS]
    y_ref, yblk_ref, w1_ref, b1_ref, w2_ref, b2_ref, o_ref = refs[_MP_STREAMS:]
    y = y_ref[...]
    ay = jnp.concatenate(
        [jnp.dot(a[...], y, preferred_element_type=jnp.float32)
         for a in adj_refs], axis=0)
    h = _leaky(jnp.dot(ay, w1_ref[...], preferred_element_type=jnp.float32) + b1_ref[...])
    g = jnp.dot(h, w2_ref[...], preferred_element_type=jnp.float32) + b2_ref[...]
    o_ref[...] = g + yblk_ref[...]


def _mp_step(adj, y, W1, b1, W2, b2, bm):
    n, h = y.shape
    ns = _MP_STREAMS
    hb = bm // ns

    def _adj_spec(j):
        return pl.BlockSpec((hb, n), lambda i: (ns * i + j, 0))

    return pl.pallas_call(
        _mp_step_body,
        grid=(n // bm,),
        in_specs=[_adj_spec(j) for j in range(ns)] + [
            pl.BlockSpec((n, h), lambda i: (0, 0)),
            pl.BlockSpec((bm, h), lambda i: (i, 0)),
            pl.BlockSpec((h, h), lambda i: (0, 0)),
            pl.BlockSpec((1, h), lambda i: (0, 0)),
            pl.BlockSpec((h, h), lambda i: (0, 0)),
            pl.BlockSpec((1, h), lambda i: (0, 0)),
        ],
        out_specs=pl.BlockSpec((bm, h), lambda i: (i, 0)),
        out_shape=jax.ShapeDtypeStruct((n, h), jnp.float32),
    )(*([adj] * ns), y, y, W1, b1.reshape(1, -1), W2, b2.reshape(1, -1))


def _post_ab_body(y_ref, wk11, bk11, wk12, bk12,
                  wk21, bk21, wk22, bk22, a_ref, b_ref):
    y = y_ref[...]
    ha = _leaky(jnp.dot(y, wk11[...], preferred_element_type=jnp.float32) + bk11[...])
    a_ref[...] = jnp.dot(ha, wk12[...], preferred_element_type=jnp.float32) + bk12[...]
    hb = _leaky(jnp.dot(y, wk21[...], preferred_element_type=jnp.float32) + bk21[...])
    b_ref[...] = jnp.dot(hb, wk22[...], preferred_element_type=jnp.float32) + bk22[...]


def _node_post_ab(y, wk11, bk11, wk12q, bk12q, wk21, bk21, wk22q, bk22q, bm):
    n, h = y.shape
    wspec = pl.BlockSpec((h, h), lambda i: (0, 0))
    bspec = pl.BlockSpec((1, h), lambda i: (0, 0))
    return pl.pallas_call(
        _post_ab_body,
        grid=(n // bm,),
        in_specs=[
            pl.BlockSpec((bm, h), lambda i: (i, 0)),
            wspec, bspec, wspec, bspec,
            wspec, bspec, wspec, bspec,
        ],
        out_specs=[
            pl.BlockSpec((bm, h), lambda i: (i, 0)),
            pl.BlockSpec((bm, h), lambda i: (i, 0)),
        ],
        out_shape=[
            jax.ShapeDtypeStruct((n, h), jnp.float32),
            jax.ShapeDtypeStruct((n, h), jnp.float32),
        ],
    )(y, wk11, bk11.reshape(1, -1), wk12q, bk12q.reshape(1, -1),
      wk21, bk21.reshape(1, -1), wk22q, bk22q.reshape(1, -1))


def _edge_body(s_ref, ef_ref, wl1, bl1, wl2, blc, wq2, bq2, o_ref):
    hl = _leaky(jnp.dot(ef_ref[...], wl1[...], preferred_element_type=jnp.float32) + bl1[...])
    c = jnp.dot(hl, wl2[...], preferred_element_type=jnp.float32) + blc[...]
    t = _leaky(s_ref[...] + c)
    o_ref[...] = jnp.dot(t, wq2[...], preferred_element_type=jnp.float32) + bq2[...]


def _edge_final(S, ef, wl1, bl1, wl2q, blc, wq2, bq2, bm, blk_off=0):
    e, h = S.shape
    e_feats = ef.shape[1]
    e_out = wq2.shape[1]
    return pl.pallas_call(
        _edge_body,
        grid=(e // bm,),
        in_specs=[
            pl.BlockSpec((bm, h), lambda i: (i, 0)),
            pl.BlockSpec((bm, e_feats), lambda i: (i + blk_off, 0)),
            pl.BlockSpec((e_feats, h), lambda i: (0, 0)),
            pl.BlockSpec((1, h), lambda i: (0, 0)),
            pl.BlockSpec((h, h), lambda i: (0, 0)),
            pl.BlockSpec((1, h), lambda i: (0, 0)),
            pl.BlockSpec((h, e_out), lambda i: (0, 0)),
            pl.BlockSpec((1, e_out), lambda i: (0, 0)),
        ],
        out_specs=pl.BlockSpec((bm, e_out), lambda i: (i, 0)),
        out_shape=jax.ShapeDtypeStruct((e, e_out), jnp.float32),
    )(S, ef, wl1, bl1.reshape(1, -1), wl2q, blc.reshape(1, -1),
      wq2, bq2.reshape(1, -1))


_NC, _NS = 2, 16
_NW = _NC * _NS


def _make_gather_combine(n, h, e, ch, eoff=0, nbuf=8):
    per_w = e // _NW
    n_chunks = per_w // ch
    assert per_w % ch == 0 and ch % 8 == 0
    mesh = plsc.VectorSubcoreMesh(core_axis_name="c", subcore_axis_name="s")

    @functools.partial(
        pl.kernel,
        mesh=mesh,
        compiler_params=pltpu.CompilerParams(use_tc_tiling_on_sc=False),
        out_type=jax.ShapeDtypeStruct((e, h), jnp.float32),
        scratch_types=[
            pltpu.VMEM((per_w,), jnp.int32),
            pltpu.VMEM((per_w,), jnp.int32),
            pltpu.VMEM((nbuf, ch, h), jnp.float32),
            pltpu.SemaphoreType.DMA((nbuf,)),
            pltpu.SemaphoreType.DMA((nbuf,)),
            pltpu.SemaphoreType.DMA((nbuf,)),
        ],
    )
    def gather_combine(a_hbm, b_hbm, e0_hbm, e1_hbm, s_hbm,
                       idx0, idx1, buf, gsem, bsem, ssem):
        wid = lax.axis_index("s") * _NC + lax.axis_index("c")
        wbase = wid * per_w
        pltpu.sync_copy(e0_hbm.at[pl.ds(eoff + wbase, per_w)], idx0)
        pltpu.sync_copy(e1_hbm.at[pl.ds(eoff + wbase, per_w)], idx1)

        def fire(c, p):
            pltpu.async_copy(a_hbm.at[idx0.at[pl.ds(c * ch, ch)]],
                             buf.at[p], gsem.at[p])

        for p in range(min(nbuf, n_chunks)):
            fire(p, p)
        for c in range(n_chunks):
            p = c % nbuf
            dst = s_hbm.at[pl.ds(wbase + c * ch, ch)]
            pltpu.make_async_copy(a_hbm.at[idx0.at[pl.ds(c * ch, ch)]],
                                  buf.at[p], gsem.at[p]).wait()
            pltpu.async_copy(b_hbm.at[idx1.at[pl.ds(c * ch, ch)]],
                             buf.at[p], bsem.at[p], add=True).wait()
            pltpu.async_copy(buf.at[p], dst, ssem.at[p])
            if c + nbuf < n_chunks:
                pltpu.make_async_copy(buf.at[p], dst, ssem.at[p]).wait()
                fire(c + nbuf, p)
        for c in range(max(0, n_chunks - nbuf), n_chunks):
            p = c % nbuf
            pltpu.make_async_copy(
                buf.at[p], s_hbm.at[pl.ds(wbase + c * ch, ch)],
                ssem.at[p]).wait()

    return gather_combine


def kernel(node_feats, adj_mat, edges, edge_feats, params):
    (wf1, bf1), (wf2, bf2) = params['f']
    (wg1, bg1), (wg2, bg2) = params['g']
    (wh1, bh1), (wh2, bh2) = params['h']
    (wk11, bk11), (wk12, bk12) = params['k1']
    (wk21, bk21), (wk22, bk22) = params['k2']
    (wl1, bl1), (wl2, bl2) = params['l']
    (wq1, bq1), (wq2, bq2) = params['q']

    n = node_feats.shape[0]
    e = edges.shape[0]
    h = wf2.shape[1]

    wk12q = wk12 @ wq1
    bk12q = bk12 @ wq1
    wk22q = wk22 @ wq1
    bk22q = bk22 @ wq1
    wl2q = wl2 @ wq1
    blc = bl2 @ wq1 + bq1

    bm_n = 1000 if n % 1000 == 0 else 8
    bm_mp = 400 if n % 400 == 0 else 8 * _MP_STREAMS
    bm_e = 8000 if e % 8000 == 0 else 8

    y = _mlp2(node_feats, wf1, bf1, wf2, bf2, bm_n)
    for _ in range(2):
        y = _mp_step(adj_mat, y, wg1, bg1, wg2, bg2, bm_mp)
    a_tab, b_tab = _node_post_ab(
        y, wk11, bk11, wk12q, bk12q, wk21, bk21, wk22q, bk22q, bm_n)

    e0 = jnp.asarray(edges[:, 0], jnp.int32)
    e1 = jnp.asarray(edges[:, 1], jnp.int32)
    n_split, e_part = 1, e
    ch = next(c for c in (200, 8) if (e // _NW) % c == 0)
    outs = []
    for si in range(n_split):
        s_i = _make_gather_combine(n, h, e_part, ch, eoff=si * e_part)(
            a_tab, b_tab, e0, e1)
        outs.append(_edge_final(s_i, edge_feats, wl1, bl1, wl2q, blc,
                                wq2, bq2, bm_e, blk_off=si * (e_part // bm_e)))

    node_outputs = _mlp2(y, wh1, bh1, wh2, bh2, bm_n)
    edge_outputs = outs[0] if n_split == 1 else jnp.concatenate(outs, axis=0)
    return (node_outputs, edge_outputs)


# speedup vs baseline: 1.1224x; 1.0111x over previous
import functools

import jax
import jax.numpy as jnp
from jax import lax
from jax.experimental import pallas as pl
from jax.experimental.pallas import tpu as pltpu
from jax.experimental.pallas import tpu_sc as plsc


def _leaky(x):
    return jnp.where(x > 0, x, 0.01 * x)


def _mlp2_body(x_ref, w1_ref, b1_ref, w2_ref, b2_ref, o_ref):
    h = jnp.dot(x_ref[...], w1_ref[...], preferred_element_type=jnp.float32)
    h = _leaky(h + b1_ref[...])
    o_ref[...] = jnp.dot(h, w2_ref[...], preferred_element_type=jnp.float32) + b2_ref[...]


def _mlp2(x, W1, b1, W2, b2, bm):
    n, d_in = x.shape
    d_mid = W1.shape[1]
    d_out = W2.shape[1]
    return pl.pallas_call(
        _mlp2_body,
        grid=(n // bm,),
        in_specs=[
            pl.BlockSpec((bm, d_in), lambda i: (i, 0)),
            pl.BlockSpec((d_in, d_mid), lambda i: (0, 0)),
            pl.BlockSpec((1, d_mid), lambda i: (0, 0)),
            pl.BlockSpec((d_mid, d_out), lambda i: (0, 0)),
            pl.BlockSpec((1, d_out), lambda i: (0, 0)),
        ],
        out_specs=pl.BlockSpec((bm, d_out), lambda i: (i, 0)),
        out_shape=jax.ShapeDtypeStruct((n, d_out), jnp.float32),
    )(x, W1, b1.reshape(1, -1), W2, b2.reshape(1, -1))


_MP_STREAMS = 2


def _mp_step_body(*refs):
    adj_refs = refs[:_MP_STREAMS]
    y_ref, yblk_ref, w1_ref, b1_ref, w2_ref, b2_ref, o_ref = refs[_MP_STREAMS:]
    y = y_ref[...]
    ay = jnp.concatenate(
        [jnp.dot(a[...], y, preferred_element_type=jnp.float32)
         for a in adj_refs], axis=0)
    h = _leaky(jnp.dot(ay, w1_ref[...], preferred_element_type=jnp.float32) + b1_ref[...])
    g = jnp.dot(h, w2_ref[...], preferred_element_type=jnp.float32) + b2_ref[...]
    o_ref[...] = g + yblk_ref[...]


def _mp_step(adj, y, W1, b1, W2, b2, bm):
    n, h = y.shape
    ns = _MP_STREAMS
    hb = bm // ns

    def _adj_spec(j):
        return pl.BlockSpec((hb, n), lambda i: (ns * i + j, 0))

    return pl.pallas_call(
        _mp_step_body,
        grid=(n // bm,),
        in_specs=[_adj_spec(j) for j in range(ns)] + [
            pl.BlockSpec((n, h), lambda i: (0, 0)),
            pl.BlockSpec((bm, h), lambda i: (i, 0)),
            pl.BlockSpec((h, h), lambda i: (0, 0)),
            pl.BlockSpec((1, h), lambda i: (0, 0)),
            pl.BlockSpec((h, h), lambda i: (0, 0)),
            pl.BlockSpec((1, h), lambda i: (0, 0)),
        ],
        out_specs=pl.BlockSpec((bm, h), lambda i: (i, 0)),
        out_shape=jax.ShapeDtypeStruct((n, h), jnp.float32),
    )(*([adj] * ns), y, y, W1, b1.reshape(1, -1), W2, b2.reshape(1, -1))


def _post_ab_body(y_ref, wk11, bk11, wk12, bk12,
                  wk21, bk21, wk22, bk22, a_ref, b_ref):
    y = y_ref[...]
    ha = _leaky(jnp.dot(y, wk11[...], preferred_element_type=jnp.float32) + bk11[...])
    a_ref[...] = jnp.dot(ha, wk12[...], preferred_element_type=jnp.float32) + bk12[...]
    hb = _leaky(jnp.dot(y, wk21[...], preferred_element_type=jnp.float32) + bk21[...])
    b_ref[...] = jnp.dot(hb, wk22[...], preferred_element_type=jnp.float32) + bk22[...]


def _node_post_ab(y, wk11, bk11, wk12q, bk12q, wk21, bk21, wk22q, bk22q, bm):
    n, h = y.shape
    wspec = pl.BlockSpec((h, h), lambda i: (0, 0))
    bspec = pl.BlockSpec((1, h), lambda i: (0, 0))
    return pl.pallas_call(
        _post_ab_body,
        grid=(n // bm,),
        in_specs=[
            pl.BlockSpec((bm, h), lambda i: (i, 0)),
            wspec, bspec, wspec, bspec,
            wspec, bspec, wspec, bspec,
        ],
        out_specs=[
            pl.BlockSpec((bm, h), lambda i: (i, 0)),
            pl.BlockSpec((bm, h), lambda i: (i, 0)),
        ],
        out_shape=[
            jax.ShapeDtypeStruct((n, h), jnp.float32),
            jax.ShapeDtypeStruct((n, h), jnp.float32),
        ],
    )(y, wk11, bk11.reshape(1, -1), wk12q, bk12q.reshape(1, -1),
      wk21, bk21.reshape(1, -1), wk22q, bk22q.reshape(1, -1))


def _edge_body(s_ref, ef_ref, wl1, bl1, wl2, blc, wq2, bq2, o_ref):
    hl = _leaky(jnp.dot(ef_ref[...], wl1[...], preferred_element_type=jnp.float32) + bl1[...])
    c = jnp.dot(hl, wl2[...], preferred_element_type=jnp.float32) + blc[...]
    t = _leaky(s_ref[...] + c)
    o_ref[...] = jnp.dot(t, wq2[...], preferred_element_type=jnp.float32) + bq2[...]


def _edge_final(S, ef, wl1, bl1, wl2q, blc, wq2, bq2, bm, blk_off=0):
    e, h = S.shape
    e_feats = ef.shape[1]
    e_out = wq2.shape[1]
    return pl.pallas_call(
        _edge_body,
        grid=(e // bm,),
        in_specs=[
            pl.BlockSpec((bm, h), lambda i: (i, 0)),
            pl.BlockSpec((bm, e_feats), lambda i: (i + blk_off, 0)),
            pl.BlockSpec((e_feats, h), lambda i: (0, 0)),
            pl.BlockSpec((1, h), lambda i: (0, 0)),
            pl.BlockSpec((h, h), lambda i: (0, 0)),
            pl.BlockSpec((1, h), lambda i: (0, 0)),
            pl.BlockSpec((h, e_out), lambda i: (0, 0)),
            pl.BlockSpec((1, e_out), lambda i: (0, 0)),
        ],
        out_specs=pl.BlockSpec((bm, e_out), lambda i: (i, 0)),
        out_shape=jax.ShapeDtypeStruct((e, e_out), jnp.float32),
    )(S, ef, wl1, bl1.reshape(1, -1), wl2q, blc.reshape(1, -1),
      wq2, bq2.reshape(1, -1))


_NC, _NS = 2, 16
_NW = _NC * _NS


def _make_gather_combine(n, h, e, ch, eoff=0, nbuf=8):
    per_w = e // _NW
    n_chunks = per_w // ch
    assert per_w % ch == 0 and ch % 8 == 0
    mesh = plsc.VectorSubcoreMesh(core_axis_name="c", subcore_axis_name="s")

    @functools.partial(
        pl.kernel,
        mesh=mesh,
        compiler_params=pltpu.CompilerParams(use_tc_tiling_on_sc=False),
        out_type=jax.ShapeDtypeStruct((e, h), jnp.float32),
        scratch_types=[
            pltpu.VMEM((per_w,), jnp.int32),
            pltpu.VMEM((per_w,), jnp.int32),
            pltpu.VMEM((nbuf, ch, h), jnp.float32),
            pltpu.SemaphoreType.DMA((nbuf,)),
            pltpu.SemaphoreType.DMA((nbuf,)),
            pltpu.SemaphoreType.DMA((nbuf,)),
        ],
    )
    def gather_combine(a_hbm, b_hbm, e0_hbm, e1_hbm, s_hbm,
                       idx0, idx1, buf, gsem, bsem, ssem):
        wid = lax.axis_index("s") * _NC + lax.axis_index("c")
        wbase = wid * per_w
        pltpu.sync_copy(e0_hbm.at[pl.ds(eoff + wbase, per_w)], idx0)
        pltpu.sync_copy(e1_hbm.at[pl.ds(eoff + wbase, per_w)], idx1)

        def fire(c, p):
            pltpu.async_copy(a_hbm.at[idx0.at[pl.ds(c * ch, ch)]],
                             buf.at[p], gsem.at[p])

        for p in range(min(nbuf, n_chunks)):
            fire(p, p)
        for c in range(n_chunks):
            p = c % nbuf
            dst = s_hbm.at[pl.ds(wbase + c * ch, ch)]
            pltpu.make_async_copy(a_hbm.at[idx0.at[pl.ds(c * ch, ch)]],
                                  buf.at[p], gsem.at[p]).wait()
            pltpu.async_copy(b_hbm.at[idx1.at[pl.ds(c * ch, ch)]],
                             buf.at[p], bsem.at[p], add=True).wait()
            pltpu.async_copy(buf.at[p], dst, ssem.at[p])
            if c + nbuf < n_chunks:
                pltpu.make_async_copy(buf.at[p], dst, ssem.at[p]).wait()
                fire(c + nbuf, p)
        for c in range(max(0, n_chunks - nbuf), n_chunks):
            p = c % nbuf
            pltpu.make_async_copy(
                buf.at[p], s_hbm.at[pl.ds(wbase + c * ch, ch)],
                ssem.at[p]).wait()

    return gather_combine


def kernel(node_feats, adj_mat, edges, edge_feats, params):
    (wf1, bf1), (wf2, bf2) = params['f']
    (wg1, bg1), (wg2, bg2) = params['g']
    (wh1, bh1), (wh2, bh2) = params['h']
    (wk11, bk11), (wk12, bk12) = params['k1']
    (wk21, bk21), (wk22, bk22) = params['k2']
    (wl1, bl1), (wl2, bl2) = params['l']
    (wq1, bq1), (wq2, bq2) = params['q']

    n = node_feats.shape[0]
    e = edges.shape[0]
    h = wf2.shape[1]

    wk12q = wk12 @ wq1
    bk12q = bk12 @ wq1
    wk22q = wk22 @ wq1
    bk22q = bk22 @ wq1
    wl2q = wl2 @ wq1
    blc = bl2 @ wq1 + bq1

    bm_n = 2000 if n % 2000 == 0 else 8
    bm_mp = 400 if n % 400 == 0 else 8 * _MP_STREAMS
    bm_e = 16000 if e % 16000 == 0 else 8

    y = _mlp2(node_feats, wf1, bf1, wf2, bf2, bm_n)
    for _ in range(2):
        y = _mp_step(adj_mat, y, wg1, bg1, wg2, bg2, bm_mp)
    a_tab, b_tab = _node_post_ab(
        y, wk11, bk11, wk12q, bk12q, wk21, bk21, wk22q, bk22q, bm_n)

    e0 = jnp.asarray(edges[:, 0], jnp.int32)
    e1 = jnp.asarray(edges[:, 1], jnp.int32)
    n_split, e_part = 1, e
    ch = next(c for c in (200, 8) if (e // _NW) % c == 0)
    outs = []
    for si in range(n_split):
        s_i = _make_gather_combine(n, h, e_part, ch, eoff=si * e_part)(
            a_tab, b_tab, e0, e1)
        outs.append(_edge_final(s_i, edge_feats, wl1, bl1, wl2q, blc,
                                wq2, bq2, bm_e, blk_off=si * (e_part // bm_e)))

    node_outputs = _mlp2(y, wh1, bh1, wh2, bh2, bm_n)
    edge_outputs = outs[0] if n_split == 1 else jnp.concatenate(outs, axis=0)
    return (node_outputs, edge_outputs)
